# Initial kernel scaffold; baseline (speedup 1.0000x reference)
import jax
import jax.numpy as jnp
from jax.experimental import pallas as pl


def kernel(x, w1, b1, w2, b2):
    raise NotImplementedError("write your pallas kernel here")


# speedup vs baseline: 2.4527x; 2.4527x over previous
import functools

import jax
import jax.numpy as jnp
from jax.experimental import pallas as pl
from jax.experimental.pallas import tpu as pltpu


def _se_fused_kernel(inv_s, x_ref, w1_ref, b1_ref, w2_ref, b2_ref, o_ref):
    x = x_ref[0]
    z = jnp.sum(x, axis=-1, keepdims=True) * inv_s
    h = jnp.dot(w1_ref[...], z, preferred_element_type=jnp.float32) + b1_ref[...]
    h = jnp.maximum(h, 0.0)
    g = jnp.dot(w2_ref[...], h, preferred_element_type=jnp.float32) + b2_ref[...]
    g = jax.nn.sigmoid(g)
    o_ref[0] = (x * g).astype(o_ref.dtype)


def kernel(x, w1, b1, w2, b2):
    B, C, D, H, W = x.shape
    Cr = w1.shape[0]
    S = D * H * W

    x3 = x.reshape(B, C, S)
    out = pl.pallas_call(
        functools.partial(_se_fused_kernel, 1.0 / float(S)),
        out_shape=jax.ShapeDtypeStruct((B, C, S), x.dtype),
        grid=(B,),
        in_specs=[
            pl.BlockSpec((1, C, S), lambda b: (b, 0, 0)),
            pl.BlockSpec((Cr, C), lambda b: (0, 0)),
            pl.BlockSpec((Cr, 1), lambda b: (0, 0)),
            pl.BlockSpec((C, Cr), lambda b: (0, 0)),
            pl.BlockSpec((C, 1), lambda b: (0, 0)),
        ],
        out_specs=pl.BlockSpec((1, C, S), lambda b: (b, 0, 0)),
        compiler_params=pltpu.CompilerParams(
            dimension_semantics=("parallel",)),
    )(x3, w1, b1.reshape(Cr, 1), w2, b2.reshape(C, 1))
    return out.reshape(B, C, D, H, W)


# speedup vs baseline: 9.0045x; 3.6712x over previous
import functools

import jax
import jax.numpy as jnp
from jax.experimental import pallas as pl
from jax.experimental.pallas import tpu as pltpu


def _se_fused_kernel(inv_s, x_ref, w1t_ref, b1_ref, w2t_ref, b2_ref, o_ref):
    x = x_ref[0]
    z = jnp.sum(x, axis=0, keepdims=True) * inv_s
    h = jnp.dot(z, w1t_ref[...], preferred_element_type=jnp.float32) + b1_ref[...]
    h = jnp.maximum(h, 0.0)
    g = jnp.dot(h, w2t_ref[...], preferred_element_type=jnp.float32) + b2_ref[...]
    g = jax.nn.sigmoid(g)
    o_ref[0] = (x * g).astype(o_ref.dtype)


def kernel(x, w1, b1, w2, b2):
    B, C, D, H, W = x.shape
    Cr = w1.shape[0]
    S = D * H * W

    xt = x.reshape(B, C, S).transpose(0, 2, 1)
    out = pl.pallas_call(
        functools.partial(_se_fused_kernel, 1.0 / float(S)),
        out_shape=jax.ShapeDtypeStruct((B, S, C), x.dtype),
        grid=(B,),
        in_specs=[
            pl.BlockSpec((1, S, C), lambda b: (b, 0, 0)),
            pl.BlockSpec((C, Cr), lambda b: (0, 0)),
            pl.BlockSpec((1, Cr), lambda b: (0, 0)),
            pl.BlockSpec((Cr, C), lambda b: (0, 0)),
            pl.BlockSpec((1, C), lambda b: (0, 0)),
        ],
        out_specs=pl.BlockSpec((1, S, C), lambda b: (b, 0, 0)),
        compiler_params=pltpu.CompilerParams(
            dimension_semantics=("parallel",)),
    )(xt, w1.T, b1.reshape(1, Cr), w2.T, b2.reshape(1, C))
    return out.transpose(0, 2, 1).reshape(B, C, D, H, W)


# speedup vs baseline: 9.3855x; 1.0423x over previous
import functools

import jax
import jax.numpy as jnp
from jax.experimental import pallas as pl
from jax.experimental.pallas import tpu as pltpu


def _se_fused_kernel(inv_s, x_ref, w1t_ref, b1_ref, w2t_ref, b2_ref, o_ref):
    x = x_ref[...]
    z = jnp.sum(x, axis=1) * inv_s
    h = jnp.dot(z, w1t_ref[...], preferred_element_type=jnp.float32) + b1_ref[...]
    h = jnp.maximum(h, 0.0)
    g = jnp.dot(h, w2t_ref[...], preferred_element_type=jnp.float32) + b2_ref[...]
    g = jax.nn.sigmoid(g)
    o_ref[...] = (x * g[:, None, :]).astype(o_ref.dtype)


def kernel(x, w1, b1, w2, b2):
    B, C, D, H, W = x.shape
    Cr = w1.shape[0]
    S = D * H * W
    bB = 2 if B % 2 == 0 else 1

    xt = x.reshape(B, C, S).transpose(0, 2, 1)
    out = pl.pallas_call(
        functools.partial(_se_fused_kernel, 1.0 / float(S)),
        out_shape=jax.ShapeDtypeStruct((B, S, C), x.dtype),
        grid=(B // bB,),
        in_specs=[
            pl.BlockSpec((bB, S, C), lambda b: (b, 0, 0)),
            pl.BlockSpec((C, Cr), lambda b: (0, 0)),
            pl.BlockSpec((1, Cr), lambda b: (0, 0)),
            pl.BlockSpec((Cr, C), lambda b: (0, 0)),
            pl.BlockSpec((1, C), lambda b: (0, 0)),
        ],
        out_specs=pl.BlockSpec((bB, S, C), lambda b: (b, 0, 0)),
        compiler_params=pltpu.CompilerParams(
            dimension_semantics=("parallel",)),
    )(xt, w1.T, b1.reshape(1, Cr), w2.T, b2.reshape(1, C))
    return out.transpose(0, 2, 1).reshape(B, C, D, H, W)
